# Initial kernel scaffold; baseline (speedup 1.0000x reference)
import jax
import jax.numpy as jnp
from jax.experimental import pallas as pl


def kernel(hist, wish, bid, auth, lang, tags, dense, book_table, auth_table, lang_table, tag_table, dw1, db1, dw2, db2, uw1, ub1, uw2, ub2, uw3, ub3, ww, wb):
    raise NotImplementedError("write your pallas kernel here")


# speedup vs baseline: 6.7484x; 6.7484x over previous
import functools

import jax
import jax.numpy as jnp
from jax import lax
from jax.experimental import pallas as pl
from jax.experimental.pallas import tpu as pltpu
from jax.experimental.pallas import tpu_sc as plsc

_B = 4096
_C = 50
_D = 128
_HIST = 200
_WISH = 50
_T = 5
_DH = 32
_H1 = 256
_H2 = 128

_CP = 64
_NC = 2
_NS = 16
_NW = _NC * _NS
_BPW = _B // _NW

_LANES = 16
_NJ = _D // _LANES


def _sc_mesh():
    return plsc.VectorSubcoreMesh(core_axis_name="c", subcore_axis_name="s")


def _user_pool_body(hist_hbm, wish_hbm, book_hbm, out_hbm,
                    hidx, widx, buf, ostg, sem):
    wid = lax.axis_index("s") * _NC + lax.axis_index("c")
    base = wid * _BPW

    def row(i, carry):
        b = base + i
        pltpu.sync_copy(hist_hbm.at[b], hidx)
        pltpu.sync_copy(wish_hbm.at[b], widx)
        cp0 = pltpu.async_copy(book_hbm.at[hidx.at[pl.ds(0, 128)]],
                               buf.at[pl.ds(0, 128)], sem)
        cp1 = pltpu.async_copy(book_hbm.at[hidx.at[pl.ds(128, 72)]],
                               buf.at[pl.ds(128, 72)], sem)
        cp2 = pltpu.async_copy(book_hbm.at[widx.at[pl.ds(0, 50)]],
                               buf.at[pl.ds(200, 50)], sem)
        cp0.wait()
        cp1.wait()
        cp2.wait()

        def acc(r, c):
            return tuple(c[j] + buf[r, pl.ds(_LANES * j, _LANES)]
                         for j in range(_NJ))

        zeros = tuple(jnp.zeros((_LANES,), jnp.float32) for _ in range(_NJ))
        hsum = lax.fori_loop(0, _HIST, acc, zeros)
        wsum = lax.fori_loop(_HIST, _HIST + _WISH, acc, zeros)
        for j in range(_NJ):
            ostg[i, pl.ds(_LANES * j, _LANES)] = (
                hsum[j] * (1.0 / _HIST) + wsum[j] * (1.0 / _WISH))
        return carry

    lax.fori_loop(0, _BPW, row, 0)
    pltpu.sync_copy(ostg, out_hbm.at[pl.ds(base, _BPW)])


def _user_pool(hist_r, wish_p, book):
    f = functools.partial(
        pl.kernel,
        out_type=jax.ShapeDtypeStruct((_B, _D), jnp.float32),
        mesh=_sc_mesh(),
        compiler_params=pltpu.CompilerParams(needs_layout_passes=False),
        scratch_types=[
            pltpu.VMEM((_HIST,), jnp.int32),
            pltpu.VMEM((56,), jnp.int32),
            pltpu.VMEM((_HIST + _WISH, _D), jnp.float32),
            pltpu.VMEM((_BPW, _D), jnp.float32),
            pltpu.SemaphoreType.DMA,
        ],
    )(_user_pool_body)
    return f(hist_r, wish_p, book)


_BT = 512


def _dense_tc_body(up_ref, d0_ref, d1_ref, d2_ref,
                   uw1t_ref, ub1_ref, uw2t_ref, ub2_ref, uw3t_ref, ub3_ref,
                   dw1_ref, db1_ref, dw2_ref, db2_ref, ww_ref, wb_ref,
                   uemb_ref, part_ref):
    up = up_ref[...]
    h = jnp.maximum(
        jnp.dot(up, uw1t_ref[...], preferred_element_type=jnp.float32)
        + ub1_ref[...][None, :], 0.0)
    h = jnp.maximum(
        jnp.dot(h, uw2t_ref[...], preferred_element_type=jnp.float32)
        + ub2_ref[...][None, :], 0.0)
    ue = (jnp.dot(h, uw3t_ref[...], preferred_element_type=jnp.float32)
          + ub3_ref[...][None, :])
    uemb_ref[...] = ue

    w = jnp.dot(ue, dw2_ref[...], preferred_element_type=jnp.float32)
    c2 = jnp.sum(ue * db2_ref[...][None, :], axis=1)

    d0 = d0_ref[...]
    d1 = d1_ref[...]
    d2 = d2_ref[...]
    acc = jnp.zeros_like(d0)
    for k in range(_DH):
        dh_k = jnp.maximum(
            d0 * dw1_ref[k, 0] + d1 * dw1_ref[k, 1] + d2 * dw1_ref[k, 2]
            + db1_ref[k], 0.0)
        acc = acc + dh_k * w[:, k][:, None]
    wide = d0 * ww_ref[0, 0] + d1 * ww_ref[0, 1] + d2 * ww_ref[0, 2]
    part_ref[...] = acc + wide + c2[:, None] + wb_ref[0]


def _dense_tc(u_pool, d0, d1, d2, uw1t, ub1, uw2t, ub2, uw3t, ub3,
              dw1, db1, dw2, db2, ww, wb):
    nblk = _B // _BT
    full = lambda shape: pl.BlockSpec(shape, lambda i: (0,) * len(shape))
    blk2 = lambda m: pl.BlockSpec((_BT, m), lambda i: (i, 0))
    return pl.pallas_call(
        _dense_tc_body,
        grid=(nblk,),
        in_specs=[
            blk2(_D), blk2(_C), blk2(_C), blk2(_C),
            full((_D, _H1)), full((_H1,)), full((_H1, _H2)), full((_H2,)),
            full((_H2, _D)), full((_D,)),
            full((_DH, 3)), full((_DH,)), full((_D, _DH)), full((_D,)),
            full((1, 3)), full((1,)),
        ],
        out_specs=[blk2(_D), blk2(_C)],
        out_shape=[
            jax.ShapeDtypeStruct((_B, _D), jnp.float32),
            jax.ShapeDtypeStruct((_B, _C), jnp.float32),
        ],
    )(u_pool, d0, d1, d2, uw1t, ub1, uw2t, ub2, uw3t, ub3,
      dw1, db1, dw2, db2, ww, wb)


def _item_body(bid_hbm, auth_hbm, lang_hbm, tags_hbm,
               book_hbm, atab_hbm, ltab_hbm, ttab_hbm,
               ue_hbm, part_hbm, out_hbm,
               bidx, aidx, lidx, tidx, buf, mat, ue, pstg, ostg, sem):
    wid = lax.axis_index("s") * _NC + lax.axis_index("c")
    base = wid * _BPW
    pltpu.sync_copy(ue_hbm.at[pl.ds(base, _BPW)], ue)
    pltpu.sync_copy(part_hbm.at[pl.ds(base, _BPW)], pstg)
    lanes = lax.iota(jnp.int32, _LANES)

    def row(i, carry):
        b = base + i
        pltpu.sync_copy(bid_hbm.at[b], bidx)
        pltpu.sync_copy(auth_hbm.at[b], aidx)
        pltpu.sync_copy(lang_hbm.at[b], lidx)
        pltpu.sync_copy(tags_hbm.at[b], tidx)
        cps = [
            pltpu.async_copy(book_hbm.at[bidx.at[pl.ds(0, _C)]],
                             buf.at[pl.ds(0, _C)], sem),
            pltpu.async_copy(atab_hbm.at[aidx.at[pl.ds(0, _C)]],
                             buf.at[pl.ds(_C, _C)], sem),
            pltpu.async_copy(ltab_hbm.at[lidx.at[pl.ds(0, _C)]],
                             buf.at[pl.ds(2 * _C, _C)], sem),
            pltpu.async_copy(ttab_hbm.at[tidx.at[pl.ds(0, 128)]],
                             buf.at[pl.ds(150, 128)], sem),
            pltpu.async_copy(ttab_hbm.at[tidx.at[pl.ds(128, 122)]],
                             buf.at[pl.ds(278, 122)], sem),
        ]
        for cp in cps:
            cp.wait()

        us = tuple(ue[i, pl.ds(_LANES * j, _LANES)] for j in range(_NJ))

        def chunk(m, carry):
            def cand(cl, carry):
                c = _LANES * m + cl
                s = jnp.zeros((_LANES,), jnp.float32)
                for j in range(_NJ):
                    sl = pl.ds(_LANES * j, _LANES)
                    v = buf[c, sl] + buf[_C + c, sl] + buf[2 * _C + c, sl]
                    t = buf[150 + 5 * c, sl]
                    for tt in range(1, _T):
                        t = t + buf[150 + 5 * c + tt, sl]
                    v = v + t * (1.0 / _T)
                    s = s + us[j] * v
                mat[cl, :] = s
                return carry

            lax.fori_loop(0, _LANES, cand, 0)
            acc = pstg[i, pl.ds(_LANES * m, _LANES)]
            for l in range(_LANES):
                acc = acc + plsc.load_gather(
                    mat, [lanes, jnp.full((_LANES,), l, jnp.int32)])
            ostg[i, pl.ds(_LANES * m, _LANES)] = acc
            return carry

        lax.fori_loop(0, _CP // _LANES, chunk, 0)
        return carry

    lax.fori_loop(0, _BPW, row, 0)
    pltpu.sync_copy(ostg, out_hbm.at[pl.ds(base, _BPW)])


def _item_score(bid_p, auth_p, lang_p, tags_p, book, atab, ltab, ttab,
                u_emb, partial):
    f = functools.partial(
        pl.kernel,
        out_type=jax.ShapeDtypeStruct((_B, _CP), jnp.float32),
        mesh=_sc_mesh(),
        compiler_params=pltpu.CompilerParams(needs_layout_passes=False),
        scratch_types=[
            pltpu.VMEM((56,), jnp.int32),
            pltpu.VMEM((56,), jnp.int32),
            pltpu.VMEM((56,), jnp.int32),
            pltpu.VMEM((256,), jnp.int32),
            pltpu.VMEM((480, _D), jnp.float32),
            pltpu.VMEM((_LANES, _LANES), jnp.float32),
            pltpu.VMEM((_BPW, _D), jnp.float32),
            pltpu.VMEM((_BPW, _CP), jnp.float32),
            pltpu.VMEM((_BPW, _CP), jnp.float32),
            pltpu.SemaphoreType.DMA,
        ],
    )(_item_body)
    return f(bid_p, auth_p, lang_p, tags_p, book, atab, ltab, ttab,
             u_emb, partial)


def _pad_idx(x, width):
    b, n = x.shape
    return jnp.concatenate(
        [x.astype(jnp.int32), jnp.zeros((b, width - n), jnp.int32)], axis=1)


def kernel(hist, wish, bid, auth, lang, tags, dense,
           book_table, auth_table, lang_table, tag_table,
           dw1, db1, dw2, db2, uw1, ub1, uw2, ub2, uw3, ub3, ww, wb):
    hist_r = hist.astype(jnp.int32)
    wish_p = _pad_idx(wish, 56)
    bid_p = _pad_idx(bid, 56)
    auth_p = _pad_idx(auth, 56)
    lang_p = _pad_idx(lang, 56)
    tags_p = _pad_idx(tags.reshape(_B, _C * _T), 256)

    u_pool = _user_pool(hist_r, wish_p, book_table)

    d0 = dense[:, :, 0]
    d1 = dense[:, :, 1]
    d2 = dense[:, :, 2]
    u_emb, partial = _dense_tc(
        u_pool, d0, d1, d2,
        uw1.T, ub1, uw2.T, ub2, uw3.T, ub3,
        dw1, db1, dw2, db2, ww, wb)

    partial_p = jnp.pad(partial, ((0, 0), (0, _CP - _C)))
    out = _item_score(bid_p, auth_p, lang_p, tags_p,
                      book_table, auth_table, lang_table, tag_table,
                      u_emb, partial_p)
    return out[:, :_C]


# speedup vs baseline: 18.4195x; 2.7295x over previous
import functools

import jax
import jax.numpy as jnp
import numpy as np
from jax import lax
from jax.experimental import pallas as pl
from jax.experimental.pallas import tpu as pltpu
from jax.experimental.pallas import tpu_sc as plsc

_B = 4096
_C = 50
_D = 128
_HIST = 200
_WISH = 50
_T = 5
_DH = 32
_H1 = 256
_H2 = 128

_CP = 64
_NC = 2
_NS = 16
_NW = _NC * _NS
_BPW = _B // _NW

_LANES = 16
_NJ = _D // _LANES

_PERM = np.concatenate([
    np.concatenate([np.arange(16 * jj, 16 * jj + 16),
                    np.arange(64 + 16 * jj, 64 + 16 * jj + 16)])
    for jj in range(_D // 32)])

_UIW = 256
_IIW = 424


def _sc_mesh():
    return plsc.VectorSubcoreMesh(core_axis_name="c", subcore_axis_name="s")


def _sc_params(tc_tiling=True):
    if tc_tiling:
        return pltpu.CompilerParams(needs_layout_passes=False)
    return pltpu.CompilerParams(needs_layout_passes=False,
                                use_tc_tiling_on_sc=False)


def _upool_copies(book_hbm, idxr, buf, sem):
    return [
        pltpu.make_async_copy(book_hbm.at[idxr.at[pl.ds(0, 128)]],
                              buf.at[pl.ds(0, 128)], sem),
        pltpu.make_async_copy(book_hbm.at[idxr.at[pl.ds(128, 72)]],
                              buf.at[pl.ds(128, 72)], sem),
        pltpu.make_async_copy(book_hbm.at[idxr.at[pl.ds(200, 50)]],
                              buf.at[pl.ds(200, 50)], sem),
    ]


def _user_pool_body(uidx_hbm, book_hbm, out_hbm,
                    idxr0, idxr1, buf0, buf1, ostg, *sems):
    idxr = (idxr0, idxr1)
    buf = (buf0, buf1)
    gsem = sems[:2]
    isem = sems[2:]
    wid = lax.axis_index("s") * _NC + lax.axis_index("c")
    base = wid * _BPW

    for p in range(2):
        pltpu.sync_copy(uidx_hbm.at[base + p], idxr[p])
        for cp in _upool_copies(book_hbm, idxr[p], buf[p], gsem[p]):
            cp.start()

    def step(g, carry):
        for p in range(2):
            i = g + p
            for cp in _upool_copies(book_hbm, idxr[p], buf[p], gsem[p]):
                cp.wait()

            @pl.when(i + 2 < _BPW)
            def _prefetch_idx():
                pltpu.make_async_copy(uidx_hbm.at[base + i + 2],
                                      idxr[p], isem[p]).start()

            bufp = buf[p]

            def acc(r, c):
                return tuple(c[j] + bufp[r, pl.ds(_LANES * j, _LANES)]
                             for j in range(_NJ))

            zeros = tuple(jnp.zeros((_LANES,), jnp.float32)
                          for _ in range(_NJ))
            hsum = lax.fori_loop(0, _HIST, acc, zeros, unroll=4)
            wsum = lax.fori_loop(_HIST, _HIST + _WISH, acc, zeros, unroll=4)
            for j in range(_NJ):
                ostg[i, pl.ds(_LANES * j, _LANES)] = (
                    hsum[j] * (1.0 / _HIST) + wsum[j] * (1.0 / _WISH))

            @pl.when(i + 2 < _BPW)
            def _prefetch_rows():
                pltpu.make_async_copy(uidx_hbm.at[base + i + 2],
                                      idxr[p], isem[p]).wait()
                for cp in _upool_copies(book_hbm, idxr[p], buf[p], gsem[p]):
                    cp.start()
        return carry

    lax.fori_loop(0, _BPW // 2, lambda g, c: step(2 * g, c), 0)
    pltpu.sync_copy(ostg, out_hbm.at[pl.ds(base, _BPW)])


def _user_pool(uidx, book):
    f = functools.partial(
        pl.kernel,
        out_type=jax.ShapeDtypeStruct((_B, _D), jnp.float32),
        mesh=_sc_mesh(),
        compiler_params=_sc_params(),
        scratch_types=[
            pltpu.VMEM((_UIW,), jnp.int32),
            pltpu.VMEM((_UIW,), jnp.int32),
            pltpu.VMEM((_HIST + _WISH, _D), jnp.float32),
            pltpu.VMEM((_HIST + _WISH, _D), jnp.float32),
            pltpu.VMEM((_BPW, _D), jnp.float32),
            pltpu.SemaphoreType.DMA,
            pltpu.SemaphoreType.DMA,
            pltpu.SemaphoreType.DMA,
            pltpu.SemaphoreType.DMA,
        ],
    )(_user_pool_body)
    return f(uidx, book)


_BT = 512


def _dense_tc_body(up_ref, d0_ref, d1_ref, d2_ref,
                   uw1t_ref, ub1_ref, uw2t_ref, ub2_ref, uw3t_ref, ub3_ref,
                   dw1_ref, db1_ref, dw2_ref, db2_ref, ww_ref, wb_ref,
                   uemb_ref, part_ref):
    up = up_ref[...]
    h = jnp.maximum(
        jnp.dot(up, uw1t_ref[...], preferred_element_type=jnp.float32)
        + ub1_ref[...][None, :], 0.0)
    h = jnp.maximum(
        jnp.dot(h, uw2t_ref[...], preferred_element_type=jnp.float32)
        + ub2_ref[...][None, :], 0.0)
    ue = (jnp.dot(h, uw3t_ref[...], preferred_element_type=jnp.float32)
          + ub3_ref[...][None, :])
    uemb_ref[...] = ue

    w = jnp.dot(ue, dw2_ref[...], preferred_element_type=jnp.float32)
    c2 = jnp.sum(ue * db2_ref[...][None, :], axis=1)

    d0 = d0_ref[...]
    d1 = d1_ref[...]
    d2 = d2_ref[...]
    acc = jnp.zeros_like(d0)
    for k in range(_DH):
        dh_k = jnp.maximum(
            d0 * dw1_ref[k, 0] + d1 * dw1_ref[k, 1] + d2 * dw1_ref[k, 2]
            + db1_ref[k], 0.0)
        acc = acc + dh_k * w[:, k][:, None]
    wide = d0 * ww_ref[0, 0] + d1 * ww_ref[0, 1] + d2 * ww_ref[0, 2]
    part_ref[...] = acc + wide + c2[:, None] + wb_ref[0]


def _dense_tc(u_pool, d0, d1, d2, uw1t, ub1, uw2t, ub2, uw3t, ub3,
              dw1, db1, dw2, db2, ww, wb):
    nblk = _B // _BT
    full = lambda shape: pl.BlockSpec(shape, lambda i: (0,) * len(shape))
    blk2 = lambda m: pl.BlockSpec((_BT, m), lambda i: (i, 0))
    return pl.pallas_call(
        _dense_tc_body,
        grid=(nblk,),
        in_specs=[
            blk2(_D), blk2(_C), blk2(_C), blk2(_C),
            full((_D, _H1)), full((_H1,)), full((_H1, _H2)), full((_H2,)),
            full((_H2, _D)), full((_D,)),
            full((_DH, 3)), full((_DH,)), full((_D, _DH)), full((_D,)),
            full((1, 3)), full((1,)),
        ],
        out_specs=[blk2(_D), blk2(_C)],
        out_shape=[
            jax.ShapeDtypeStruct((_B, _D), jnp.float32),
            jax.ShapeDtypeStruct((_B, _C), jnp.float32),
        ],
    )(u_pool, d0, d1, d2, uw1t, ub1, uw2t, ub2, uw3t, ub3,
      dw1, db1, dw2, db2, ww, wb)


def _item_copies(book_hbm, atab_hbm, ltab_hbm, idxr, bbuf, abuf, sem):
    return [
        pltpu.make_async_copy(book_hbm.at[idxr.at[pl.ds(0, _C)]],
                              bbuf.at[pl.ds(0, _C)], sem),
        pltpu.make_async_copy(atab_hbm.at[idxr.at[pl.ds(56, _C)]],
                              abuf.at[pl.ds(0, _C)], sem),
        pltpu.make_async_copy(ltab_hbm.at[idxr.at[pl.ds(112, _C)]],
                              abuf.at[pl.ds(_C, _C)], sem),
    ]


_INB = 2


def _item_body(iidx_hbm, book_hbm, atab_hbm, ltab_hbm, ttab_hbm,
               ue_hbm, part_hbm, out_hbm,
               idxr0, idxr1, bbuf0, bbuf1, abuf0, abuf1,
               uer0, uer1, tidr0, tidr1, tagv, mat, ostg, *sems):
    idxr = (idxr0, idxr1)
    bbuf = (bbuf0, bbuf1)
    abuf = (abuf0, abuf1)
    uer = (uer0, uer1)
    tidr = (tidr0, tidr1)
    gsem = sems[:_INB]
    isem = sems[_INB:2 * _INB]
    tsem = sems[2 * _INB:]
    wid = lax.axis_index("s") * _NC + lax.axis_index("c")
    base = wid * _BPW
    pltpu.sync_copy(ttab_hbm, tagv)
    pltpu.sync_copy(part_hbm.at[pl.ds(base, _BPW)], ostg)
    lanes = lax.iota(jnp.int32, _LANES)
    cols = [lanes + _LANES * jj for jj in range(_D // 32)]

    for p in range(_INB):
        pltpu.sync_copy(iidx_hbm.at[base + p], idxr[p])
        pltpu.make_async_copy(iidx_hbm.at[base + p, pl.ds(168, 256)],
                              tidr[p], tsem[p]).start()
        pltpu.sync_copy(ue_hbm.at[base + p], uer[p])
        for cp in _item_copies(book_hbm, atab_hbm, ltab_hbm,
                               idxr[p], bbuf[p], abuf[p], gsem[p]):
            cp.start()

    def step(g, carry):
        for p in range(_INB):
            i = g + p
            for cp in _item_copies(book_hbm, atab_hbm, ltab_hbm,
                                   idxr[p], bbuf[p], abuf[p], gsem[p]):
                cp.wait()
            pltpu.make_async_copy(
                iidx_hbm.at[base + i, pl.ds(168, 256)], tidr[p],
                tsem[p]).wait()
            uerp = uer[p]
            us_p = tuple(uerp[pl.ds(_LANES * j, _LANES)]
                         for j in range(_NJ))
            us_o = tuple(uerp[pl.ds(_D + _LANES * j, _LANES)]
                         for j in range(_NJ))

            @pl.when(i + _INB < _BPW)
            def _prefetch_idx():
                pltpu.make_async_copy(iidx_hbm.at[base + i + _INB],
                                      idxr[p], isem[p]).start()
                pltpu.make_async_copy(ue_hbm.at[base + i + _INB],
                                      uer[p], isem[p]).start()

            def cand_range(m, lo, hi):
                bbufp = bbuf[p]
                abufp = abuf[p]
                tidrp = tidr[p]

                def cand(cl, carry):
                    c = _LANES * m + cl
                    s = jnp.zeros((_LANES,), jnp.float32)
                    for j in range(_NJ):
                        s = s + us_o[j] * bbufp[c, pl.ds(_LANES * j, _LANES)]
                    tids = [plsc.load_gather(
                        tidrp, [jnp.full((_LANES,), 5 * c + tt, jnp.int32)])
                        for tt in range(_T)]
                    for jj in range(_D // 32):
                        sl = pl.ds(_LANES * jj, _LANES)
                        bc = lambda r: plsc.bitcast(abufp[r, sl], jnp.bfloat16)
                        tg = lambda tt: plsc.bitcast(
                            plsc.load_gather(tagv, [tids[tt], cols[jj]]),
                            jnp.bfloat16)
                        t = tg(0)
                        for tt in range(1, _T):
                            t = t + tg(tt)
                        t_lo, t_hi = plsc.unpack(
                            t, format=plsc.PackFormat.INTERLEAVED)
                        a_lo, a_hi = plsc.unpack(
                            bc(c), format=plsc.PackFormat.INTERLEAVED)
                        l_lo, l_hi = plsc.unpack(
                            bc(_C + c), format=plsc.PackFormat.INTERLEAVED)
                        v_lo = a_lo + l_lo + t_lo * (1.0 / _T)
                        v_hi = a_hi + l_hi + t_hi * (1.0 / _T)
                        s = s + us_p[2 * jj] * v_lo
                        s = s + us_p[2 * jj + 1] * v_hi
                    mat[cl, :] = s
                    return carry

                lax.fori_loop(lo, hi, cand, 0, unroll=2)
                col = lambda l: plsc.load_gather(
                    mat, [lanes, jnp.full((_LANES,), l, jnp.int32)])
                accs = [ostg[i, pl.ds(_LANES * m, _LANES)]] + [
                    jnp.zeros((_LANES,), jnp.float32) for _ in range(3)]
                for l in range(_LANES):
                    accs[l % 4] = accs[l % 4] + col(l)
                ostg[i, pl.ds(_LANES * m, _LANES)] = (
                    (accs[0] + accs[1]) + (accs[2] + accs[3]))

            for m in range(_C // _LANES):
                cand_range(m, 0, _LANES)
            cand_range(_C // _LANES, 0, _C % _LANES)

            @pl.when(i + _INB < _BPW)
            def _prefetch_rows():
                pltpu.make_async_copy(iidx_hbm.at[base + i + _INB],
                                      idxr[p], isem[p]).wait()
                pltpu.make_async_copy(ue_hbm.at[base + i + _INB],
                                      uer[p], isem[p]).wait()
                pltpu.make_async_copy(
                    iidx_hbm.at[base + i + _INB, pl.ds(168, 256)], tidr[p],
                    tsem[p]).start()
                for cp in _item_copies(book_hbm, atab_hbm, ltab_hbm,
                                       idxr[p], bbuf[p], abuf[p], gsem[p]):
                    cp.start()
        return carry

    lax.fori_loop(0, _BPW // _INB, lambda g, c: step(_INB * g, c), 0)
    pltpu.sync_copy(ostg, out_hbm.at[pl.ds(base, _BPW)])


def _item_score(iidx, book, atab, ltab, ttab, u_emb, partial):
    f = functools.partial(
        pl.kernel,
        out_type=jax.ShapeDtypeStruct((_B, _CP), jnp.float32),
        mesh=_sc_mesh(),
        compiler_params=_sc_params(tc_tiling=False),
        scratch_types=(
            [pltpu.VMEM((_IIW,), jnp.int32)] * _INB
            + [pltpu.VMEM((_C, _D), jnp.float32)] * _INB
            + [pltpu.VMEM((2 * _C, _D // 2), jnp.int32)] * _INB
            + [pltpu.VMEM((2 * _D,), jnp.float32)] * _INB
            + [pltpu.VMEM((256,), jnp.int32)] * _INB
            + [pltpu.VMEM((1001, _D // 2), jnp.int32),
               pltpu.VMEM((_LANES, _LANES), jnp.float32),
               pltpu.VMEM((_BPW, _CP), jnp.float32)]
            + [pltpu.SemaphoreType.DMA] * (3 * _INB)
        ),
    )(_item_body)
    return f(iidx, book, atab, ltab, ttab, u_emb, partial)


def _i32(x):
    return x.astype(jnp.int32)


def kernel(hist, wish, bid, auth, lang, tags, dense,
           book_table, auth_table, lang_table, tag_table,
           dw1, db1, dw2, db2, uw1, ub1, uw2, ub2, uw3, ub3, ww, wb):
    z6 = jnp.zeros((_B, 6), jnp.int32)
    uidx = jnp.concatenate([_i32(hist), _i32(wish), z6], axis=1)
    iidx = jnp.concatenate(
        [_i32(bid), z6, _i32(auth), z6, _i32(lang), z6,
         _i32(tags.reshape(_B, _C * _T)), z6], axis=1)

    def _tbl_i32(t):
        xi = lax.bitcast_convert_type(t, jnp.uint32)
        r = (xi + jnp.uint32(0x7FFF) + ((xi >> 16) & jnp.uint32(1))) >> 16
        packed = r[:, :_D // 2] | (r[:, _D // 2:] << 16)
        return lax.bitcast_convert_type(packed, jnp.int32)

    auth16 = _tbl_i32(auth_table)
    lang16 = _tbl_i32(lang_table)
    tag16 = _tbl_i32(tag_table)

    u_pool = _user_pool(uidx, book_table)

    d0 = dense[:, :, 0]
    d1 = dense[:, :, 1]
    d2 = dense[:, :, 2]
    u_emb, partial = _dense_tc(
        u_pool, d0, d1, d2,
        uw1.T, ub1, uw2.T, ub2, uw3.T, ub3,
        dw1, db1, dw2, db2, ww, wb)

    ue_both = jnp.concatenate([u_emb[:, _PERM], u_emb], axis=1)
    partial_p = jnp.pad(partial, ((0, 0), (0, _CP - _C)))
    out = _item_score(iidx, book_table, auth16, lang16, tag16,
                      ue_both, partial_p)
    return out[:, :_C]
